# Initial kernel scaffold; baseline (speedup 1.0000x reference)
import jax
import jax.numpy as jnp
from jax.experimental import pallas as pl


def kernel(input_data, pos, token_table, pos_table, W, b, gamma, beta):
    raise NotImplementedError("write your pallas kernel here")


# speedup vs baseline: 1.6093x; 1.6093x over previous
import functools

import jax
import jax.numpy as jnp
from jax import lax
from jax.experimental import pallas as pl
from jax.experimental.pallas import tpu as pltpu
from jax.experimental.pallas import tpu_sc as plsc

NC, NS = 2, 16
NW = NC * NS


def _sc_gather(table, idx, chunk):
    n = idx.shape[0]
    e = table.shape[1]
    per_w = n // NW
    n_chunks = per_w // chunk
    mesh = plsc.VectorSubcoreMesh(core_axis_name="c", subcore_axis_name="s")

    @functools.partial(
        pl.kernel,
        mesh=mesh,
        out_type=jax.ShapeDtypeStruct((n, e), jnp.float32),
        scratch_types=[
            pltpu.VMEM((chunk,), jnp.int32),
            pltpu.VMEM((chunk, e), jnp.float32),
            pltpu.SemaphoreType.DMA,
        ],
        compiler_params=pltpu.CompilerParams(use_tc_tiling_on_sc=False),
    )
    def gather_k(table_hbm, idx_hbm, out_hbm, idx_v, rows_v, sem):
        wid = lax.axis_index("s") * NC + lax.axis_index("c")
        base = wid * per_w

        def body(i, carry):
            off = base + i * chunk
            pltpu.sync_copy(idx_hbm.at[pl.ds(off, chunk)], idx_v)
            pltpu.async_copy(table_hbm.at[idx_v], rows_v, sem).wait()
            pltpu.sync_copy(rows_v, out_hbm.at[pl.ds(off, chunk)])
            return carry

        lax.fori_loop(0, n_chunks, body, 0, unroll=False)

    return gather_k(table, idx)


def _tc_finish(tok, pos_i, pos_table, W, b, gamma, beta, blk):
    n, e = tok.shape
    l = pos_table.shape[0]
    h = W.shape[0]
    grid = n // blk
    pos3 = pos_i.reshape(grid, 1, blk)
    b2 = b.reshape(1, h)
    g2 = gamma.reshape(1, h)
    be2 = beta.reshape(1, h)

    def body(tok_ref, pos_ref, ptab_ref, w_ref, b_ref, g_ref, be_ref,
             ln_ref, out_ref):
        tok_b = tok_ref[...]
        p = pos_ref[0, 0, :]
        oh = (p[:, None] == lax.broadcasted_iota(jnp.int32, (blk, l), 1))
        pe = jnp.dot(oh.astype(jnp.float32), ptab_ref[...],
                     preferred_element_type=jnp.float32)
        x = tok_b + pe
        y = lax.dot_general(x, w_ref[...], (((1,), (1,)), ((), ())),
                            preferred_element_type=jnp.float32) + b_ref[...]
        out_ref[...] = y
        mean = jnp.mean(y, axis=1, keepdims=True)
        var = jnp.mean((y - mean) ** 2, axis=1, keepdims=True)
        ln_ref[...] = (y - mean) * lax.rsqrt(var + 1e-5) * g_ref[...] + be_ref[...]

    ln, out = pl.pallas_call(
        body,
        grid=(grid,),
        in_specs=[
            pl.BlockSpec((blk, e), lambda i: (i, 0)),
            pl.BlockSpec((1, 1, blk), lambda i: (i, 0, 0)),
            pl.BlockSpec((l, e), lambda i: (0, 0)),
            pl.BlockSpec((h, e), lambda i: (0, 0)),
            pl.BlockSpec((1, h), lambda i: (0, 0)),
            pl.BlockSpec((1, h), lambda i: (0, 0)),
            pl.BlockSpec((1, h), lambda i: (0, 0)),
        ],
        out_specs=[
            pl.BlockSpec((blk, h), lambda i: (i, 0)),
            pl.BlockSpec((blk, h), lambda i: (i, 0)),
        ],
        out_shape=[
            jax.ShapeDtypeStruct((n, h), jnp.float32),
            jax.ShapeDtypeStruct((n, h), jnp.float32),
        ],
    )(tok, pos3, pos_table, W, b2, g2, be2)
    return ln, out


def kernel(input_data, pos, token_table, pos_table, W, b, gamma, beta):
    B, S = input_data.shape
    V, E = token_table.shape
    H = W.shape[0]
    n = B * S
    idx_flat = input_data.reshape(n).astype(jnp.int32)
    pos_flat = pos.reshape(n).astype(jnp.int32)

    tok = _sc_gather(token_table, idx_flat, chunk=1024)
    ln, out = _tc_finish(tok, pos_flat, pos_table, W, b, gamma, beta, blk=1024)
    return ln.reshape(B, S, H), out.reshape(B, S, H)


# speedup vs baseline: 1.6528x; 1.0270x over previous
import functools

import jax
import jax.numpy as jnp
from jax import lax
from jax.experimental import pallas as pl
from jax.experimental.pallas import tpu as pltpu
from jax.experimental.pallas import tpu_sc as plsc

NC, NS = 2, 16
NW = NC * NS


def _sc_gather(table, idx, chunk):
    n = idx.shape[0]
    e = table.shape[1]
    per_w = n // NW
    n_chunks = per_w // chunk
    mesh = plsc.VectorSubcoreMesh(core_axis_name="c", subcore_axis_name="s")

    @functools.partial(
        pl.kernel,
        mesh=mesh,
        out_type=jax.ShapeDtypeStruct((n, e), jnp.float32),
        scratch_types=[
            pltpu.VMEM((chunk,), jnp.int32),
            pltpu.VMEM((chunk, e), jnp.float32),
            pltpu.SemaphoreType.DMA,
        ],
        compiler_params=pltpu.CompilerParams(use_tc_tiling_on_sc=False),
    )
    def gather_k(table_hbm, idx_hbm, out_hbm, idx_v, rows_v, sem):
        wid = lax.axis_index("s") * NC + lax.axis_index("c")
        base = wid * per_w

        def body(i, carry):
            off = base + i * chunk
            pltpu.sync_copy(idx_hbm.at[pl.ds(off, chunk)], idx_v)
            pltpu.async_copy(table_hbm.at[idx_v], rows_v, sem).wait()
            pltpu.sync_copy(rows_v, out_hbm.at[pl.ds(off, chunk)])
            return carry

        lax.fori_loop(0, n_chunks, body, 0, unroll=False)

    return gather_k(table, idx)


def _tc_finish(tok, pos_i, pos_table, W, b, gamma, beta, blk):
    n, e = tok.shape
    l = pos_table.shape[0]
    h = W.shape[0]
    grid = n // blk
    pos3 = pos_i.reshape(grid, 1, blk)
    b2 = b.reshape(1, h)
    g2 = gamma.reshape(1, h)
    be2 = beta.reshape(1, h)

    def body(tok_ref, pos_ref, ptab_ref, w_ref, b_ref, g_ref, be_ref,
             ln_ref, out_ref):
        tok_b = tok_ref[...]
        p = pos_ref[0, 0, :]
        oh = (p[:, None] == lax.broadcasted_iota(jnp.int32, (blk, l), 1))
        pe = jnp.dot(oh.astype(jnp.float32), ptab_ref[...],
                     preferred_element_type=jnp.float32)
        x = tok_b + pe
        y = lax.dot_general(x, w_ref[...], (((1,), (1,)), ((), ())),
                            preferred_element_type=jnp.float32) + b_ref[...]
        out_ref[...] = y
        mean = jnp.mean(y, axis=1, keepdims=True)
        var = jnp.mean((y - mean) ** 2, axis=1, keepdims=True)
        ln_ref[...] = (y - mean) * lax.rsqrt(var + 1e-5) * g_ref[...] + be_ref[...]

    ln, out = pl.pallas_call(
        body,
        grid=(grid,),
        in_specs=[
            pl.BlockSpec((blk, e), lambda i: (i, 0)),
            pl.BlockSpec((1, 1, blk), lambda i: (i, 0, 0)),
            pl.BlockSpec((l, e), lambda i: (0, 0)),
            pl.BlockSpec((h, e), lambda i: (0, 0)),
            pl.BlockSpec((1, h), lambda i: (0, 0)),
            pl.BlockSpec((1, h), lambda i: (0, 0)),
            pl.BlockSpec((1, h), lambda i: (0, 0)),
        ],
        out_specs=[
            pl.BlockSpec((blk, h), lambda i: (i, 0)),
            pl.BlockSpec((blk, h), lambda i: (i, 0)),
        ],
        out_shape=[
            jax.ShapeDtypeStruct((n, h), jnp.float32),
            jax.ShapeDtypeStruct((n, h), jnp.float32),
        ],
    )(tok, pos3, pos_table, W, b2, g2, be2)
    return ln, out


def kernel(input_data, pos, token_table, pos_table, W, b, gamma, beta):
    B, S = input_data.shape
    V, E = token_table.shape
    H = W.shape[0]
    n = B * S
    idx_flat = input_data.reshape(n).astype(jnp.int32)
    pos_flat = pos.reshape(n).astype(jnp.int32)

    t2 = jnp.pad(token_table, ((0, 0), (0, 64))).reshape(2 * V, E)
    tok = _sc_gather(t2, idx_flat * 2, chunk=1024)
    ln, out = _tc_finish(tok, pos_flat, pos_table, W, b, gamma, beta, blk=1024)
    return ln.reshape(B, S, H), out.reshape(B, S, H)


# speedup vs baseline: 1.8794x; 1.1371x over previous
import functools

import jax
import jax.numpy as jnp
from jax import lax
from jax.experimental import pallas as pl
from jax.experimental.pallas import tpu as pltpu
from jax.experimental.pallas import tpu_sc as plsc

NC, NS = 2, 16
NW = NC * NS


def _sc_gather(table, idx, chunk):
    n = idx.shape[0]
    e = table.shape[1]
    per_w = n // NW
    n_chunks = per_w // chunk
    mesh = plsc.VectorSubcoreMesh(core_axis_name="c", subcore_axis_name="s")

    @functools.partial(
        pl.kernel,
        mesh=mesh,
        out_type=jax.ShapeDtypeStruct((n, e), jnp.float32),
        scratch_types=[
            pltpu.VMEM((chunk,), jnp.int32),
            pltpu.VMEM((chunk, e), jnp.float32),
            pltpu.SemaphoreType.DMA,
        ],
        compiler_params=pltpu.CompilerParams(use_tc_tiling_on_sc=False),
    )
    def gather_k(table_hbm, idx_hbm, out_hbm, idx_v, rows_v, sem):
        wid = lax.axis_index("s") * NC + lax.axis_index("c")
        base = wid * per_w

        def body(i, carry):
            off = base + i * chunk
            pltpu.sync_copy(idx_hbm.at[pl.ds(off, chunk)], idx_v)
            pltpu.async_copy(table_hbm.at[idx_v], rows_v, sem).wait()
            pltpu.sync_copy(rows_v, out_hbm.at[pl.ds(off, chunk)])
            return carry

        lax.fori_loop(0, n_chunks, body, 0, unroll=False)

    return gather_k(table, idx)


def _tc_finish(tok, pos_i, pos_table, W, b, gamma, beta, blk):
    n, e = tok.shape
    l = pos_table.shape[0]
    h = W.shape[0]
    grid = n // blk
    pos3 = pos_i.reshape(grid, 1, blk)
    b2 = b.reshape(1, h)
    g2 = gamma.reshape(1, h)
    be2 = beta.reshape(1, h)

    def body(tok_ref, pos_ref, ptab_ref, w_ref, b_ref, g_ref, be_ref,
             ln_ref, out_ref):
        tok_b = tok_ref[...]
        p = pos_ref[0, 0, :]
        oh = (p[:, None] == lax.broadcasted_iota(jnp.int32, (blk, l), 1))
        pe = jnp.dot(oh.astype(jnp.float32), ptab_ref[...],
                     preferred_element_type=jnp.float32)
        x = tok_b + pe
        y = lax.dot_general(x, w_ref[...], (((1,), (1,)), ((), ())),
                            preferred_element_type=jnp.float32) + b_ref[...]
        out_ref[...] = y
        mean = jnp.mean(y, axis=1, keepdims=True)
        var = jnp.mean((y - mean) ** 2, axis=1, keepdims=True)
        ln_ref[...] = (y - mean) * lax.rsqrt(var + 1e-5) * g_ref[...] + be_ref[...]

    ln, out = pl.pallas_call(
        body,
        grid=(grid,),
        in_specs=[
            pl.BlockSpec((blk, e), lambda i: (i, 0)),
            pl.BlockSpec((1, 1, blk), lambda i: (i, 0, 0)),
            pl.BlockSpec((l, e), lambda i: (0, 0)),
            pl.BlockSpec((h, e), lambda i: (0, 0)),
            pl.BlockSpec((1, h), lambda i: (0, 0)),
            pl.BlockSpec((1, h), lambda i: (0, 0)),
            pl.BlockSpec((1, h), lambda i: (0, 0)),
        ],
        out_specs=[
            pl.BlockSpec((blk, h), lambda i: (i, 0)),
            pl.BlockSpec((blk, h), lambda i: (i, 0)),
        ],
        out_shape=[
            jax.ShapeDtypeStruct((n, h), jnp.float32),
            jax.ShapeDtypeStruct((n, h), jnp.float32),
        ],
    )(tok, pos3, pos_table, W, b2, g2, be2)
    return ln, out


def kernel(input_data, pos, token_table, pos_table, W, b, gamma, beta):
    B, S = input_data.shape
    V, E = token_table.shape
    H = W.shape[0]
    n = B * S
    idx_flat = input_data.reshape(n).astype(jnp.int32)
    pos_flat = pos.reshape(n).astype(jnp.int32)

    t2 = jnp.pad(token_table, ((0, 0), (0, 64))).reshape(2 * V, E)
    tok = _sc_gather(t2, idx_flat * 2, chunk=1024)
    ln, out = _tc_finish(tok, pos_flat, pos_table, W, b, gamma, beta, blk=2048)
    return ln.reshape(B, S, H), out.reshape(B, S, H)


# speedup vs baseline: 2.0081x; 1.0685x over previous
import functools

import jax
import jax.numpy as jnp
from jax import lax
from jax.experimental import pallas as pl
from jax.experimental.pallas import tpu as pltpu
from jax.experimental.pallas import tpu_sc as plsc

NC, NS = 2, 16
NW = NC * NS


def _sc_gather(table, idx, chunk):
    n = idx.shape[0]
    e = table.shape[1]
    per_w = n // NW
    n_chunks = per_w // chunk
    mesh = plsc.VectorSubcoreMesh(core_axis_name="c", subcore_axis_name="s")

    @functools.partial(
        pl.kernel,
        mesh=mesh,
        out_type=jax.ShapeDtypeStruct((n, e), jnp.float32),
        scratch_types=[
            pltpu.VMEM((chunk,), jnp.int32),
            pltpu.VMEM((chunk, e), jnp.float32),
            pltpu.SemaphoreType.DMA,
        ],
        compiler_params=pltpu.CompilerParams(use_tc_tiling_on_sc=False),
    )
    def gather_k(table_hbm, idx_hbm, out_hbm, idx_v, rows_v, sem):
        wid = lax.axis_index("s") * NC + lax.axis_index("c")
        base = wid * per_w

        def body(i, carry):
            off = base + i * chunk
            pltpu.sync_copy(idx_hbm.at[pl.ds(off, chunk)], idx_v)
            pltpu.async_copy(table_hbm.at[idx_v], rows_v, sem).wait()
            pltpu.sync_copy(rows_v, out_hbm.at[pl.ds(off, chunk)])
            return carry

        lax.fori_loop(0, n_chunks, body, 0, unroll=False)

    return gather_k(table, idx)


def _tc_finish(tok, pos_i, pos_table, W, b, gamma, beta, blk):
    n, e = tok.shape
    l = pos_table.shape[0]
    h = W.shape[0]
    grid = n // blk
    pos3 = pos_i.reshape(grid, 1, blk)
    b2 = b.reshape(1, h)
    g2 = gamma.reshape(1, h)
    be2 = beta.reshape(1, h)

    def body(tok_ref, pos_ref, ptab_ref, w_ref, b_ref, g_ref, be_ref,
             ln_ref, out_ref):
        tok_b = tok_ref[...]
        p = pos_ref[0, 0, :]
        oh = (p[:, None] == lax.broadcasted_iota(jnp.int32, (blk, l), 1))
        pe = jnp.dot(oh.astype(jnp.float32), ptab_ref[...],
                     preferred_element_type=jnp.float32)
        x = tok_b + pe
        y = lax.dot_general(x, w_ref[...], (((1,), (1,)), ((), ())),
                            preferred_element_type=jnp.float32) + b_ref[...]
        out_ref[...] = y
        mean = jnp.mean(y, axis=1, keepdims=True)
        var = jnp.mean((y - mean) ** 2, axis=1, keepdims=True)
        ln_ref[...] = (y - mean) * lax.rsqrt(var + 1e-5) * g_ref[...] + be_ref[...]

    ln, out = pl.pallas_call(
        body,
        grid=(grid,),
        in_specs=[
            pl.BlockSpec((blk, e), lambda i: (i, 0)),
            pl.BlockSpec((1, 1, blk), lambda i: (i, 0, 0)),
            pl.BlockSpec((l, e), lambda i: (0, 0)),
            pl.BlockSpec((h, e), lambda i: (0, 0)),
            pl.BlockSpec((1, h), lambda i: (0, 0)),
            pl.BlockSpec((1, h), lambda i: (0, 0)),
            pl.BlockSpec((1, h), lambda i: (0, 0)),
        ],
        out_specs=[
            pl.BlockSpec((blk, h), lambda i: (i, 0)),
            pl.BlockSpec((blk, h), lambda i: (i, 0)),
        ],
        out_shape=[
            jax.ShapeDtypeStruct((n, h), jnp.float32),
            jax.ShapeDtypeStruct((n, h), jnp.float32),
        ],
    )(tok, pos3, pos_table, W, b2, g2, be2)
    return ln, out


def kernel(input_data, pos, token_table, pos_table, W, b, gamma, beta):
    B, S = input_data.shape
    V, E = token_table.shape
    H = W.shape[0]
    n = B * S
    idx_flat = input_data.reshape(n).astype(jnp.int32)
    pos_flat = pos.reshape(n).astype(jnp.int32)

    t2 = jnp.pad(token_table, ((0, 0), (0, 64))).reshape(2 * V, E)
    tok = _sc_gather(t2, idx_flat * 2, chunk=1024)
    ln, out = _tc_finish(tok, pos_flat, pos_table, W, b, gamma, beta, blk=4096)
    return ln.reshape(B, S, H), out.reshape(B, S, H)


# speedup vs baseline: 2.0881x; 1.0398x over previous
import functools

import jax
import jax.numpy as jnp
from jax import lax
from jax.experimental import pallas as pl
from jax.experimental.pallas import tpu as pltpu
from jax.experimental.pallas import tpu_sc as plsc

NC, NS = 2, 16
NW = NC * NS


def _sc_gather(table, idx, chunk):
    n = idx.shape[0]
    e = table.shape[1]
    per_w = n // NW
    n_chunks = per_w // chunk
    mesh = plsc.VectorSubcoreMesh(core_axis_name="c", subcore_axis_name="s")

    @functools.partial(
        pl.kernel,
        mesh=mesh,
        out_type=jax.ShapeDtypeStruct((n, e), jnp.float32),
        scratch_types=[
            pltpu.VMEM((chunk,), jnp.int32),
            pltpu.VMEM((chunk, e), jnp.float32),
            pltpu.SemaphoreType.DMA,
        ],
        compiler_params=pltpu.CompilerParams(use_tc_tiling_on_sc=False),
    )
    def gather_k(table_hbm, idx_hbm, out_hbm, idx_v, rows_v, sem):
        wid = lax.axis_index("s") * NC + lax.axis_index("c")
        base = wid * per_w

        def body(i, carry):
            off = base + i * chunk
            pltpu.sync_copy(idx_hbm.at[pl.ds(off, chunk)], idx_v)
            pltpu.async_copy(table_hbm.at[idx_v], rows_v, sem).wait()
            pltpu.sync_copy(rows_v, out_hbm.at[pl.ds(off, chunk)])
            return carry

        lax.fori_loop(0, n_chunks, body, 0, unroll=False)

    return gather_k(table, idx)


def _tc_finish(tok, pos_i, pos_table, W, b, gamma, beta, blk):
    n, e = tok.shape
    l = pos_table.shape[0]
    h = W.shape[0]
    grid = n // blk
    pos3 = pos_i.reshape(grid, 1, blk)
    b2 = b.reshape(1, h)
    g2 = gamma.reshape(1, h)
    be2 = beta.reshape(1, h)

    def body(tok_ref, pos_ref, ptab_ref, w_ref, b_ref, g_ref, be_ref,
             ln_ref, out_ref):
        tok_b = tok_ref[...]
        p = pos_ref[0, 0, :]
        oh = (p[:, None] == lax.broadcasted_iota(jnp.int32, (blk, l), 1))
        pe = jnp.dot(oh.astype(jnp.float32), ptab_ref[...],
                     preferred_element_type=jnp.float32)
        x = tok_b + pe
        y = lax.dot_general(x, w_ref[...], (((1,), (1,)), ((), ())),
                            preferred_element_type=jnp.float32) + b_ref[...]
        out_ref[...] = y
        mean = jnp.mean(y, axis=1, keepdims=True)
        var = jnp.mean((y - mean) ** 2, axis=1, keepdims=True)
        ln_ref[...] = (y - mean) * lax.rsqrt(var + 1e-5) * g_ref[...] + be_ref[...]

    ln, out = pl.pallas_call(
        body,
        grid=(grid,),
        in_specs=[
            pl.BlockSpec((blk, e), lambda i: (i, 0)),
            pl.BlockSpec((1, 1, blk), lambda i: (i, 0, 0)),
            pl.BlockSpec((l, e), lambda i: (0, 0)),
            pl.BlockSpec((h, e), lambda i: (0, 0)),
            pl.BlockSpec((1, h), lambda i: (0, 0)),
            pl.BlockSpec((1, h), lambda i: (0, 0)),
            pl.BlockSpec((1, h), lambda i: (0, 0)),
        ],
        out_specs=[
            pl.BlockSpec((blk, h), lambda i: (i, 0)),
            pl.BlockSpec((blk, h), lambda i: (i, 0)),
        ],
        out_shape=[
            jax.ShapeDtypeStruct((n, h), jnp.float32),
            jax.ShapeDtypeStruct((n, h), jnp.float32),
        ],
    )(tok, pos3, pos_table, W, b2, g2, be2)
    return ln, out


def kernel(input_data, pos, token_table, pos_table, W, b, gamma, beta):
    B, S = input_data.shape
    V, E = token_table.shape
    H = W.shape[0]
    n = B * S
    idx_flat = input_data.reshape(n).astype(jnp.int32)
    pos_flat = pos.reshape(n).astype(jnp.int32)

    t2 = jnp.pad(token_table, ((0, 0), (0, 64))).reshape(2 * V, E)
    tok = _sc_gather(t2, idx_flat * 2, chunk=1024)
    ln, out = _tc_finish(tok, pos_flat, pos_table, W, b, gamma, beta, blk=8192)
    return ln.reshape(B, S, H), out.reshape(B, S, H)
